# baseline (speedup 1.0000x reference)
import functools

import jax
import jax.numpy as jnp
from jax import lax
from jax.experimental import pallas as pl
from jax.experimental.pallas import tpu as pltpu
from jax.experimental.pallas import tpu_sc as plsc

_NT = 7300
_NG = 2000
_NCH = 3
_DAYS = 365
_NY = _NT // _DAYS
_COLS = _NG * _NCH
_W = 48
_NSTRIP = _COLS // _W
_NTASK = _NY * _NSTRIP
_NWORK = 32
_TPW = -(-_NTASK // _NWORK)
_RCHUNK = 5


def _sc_partials(o2, t2):
    mesh = plsc.VectorSubcoreMesh(core_axis_name="c", subcore_axis_name="s")

    @functools.partial(
        pl.kernel,
        out_type=jax.ShapeDtypeStruct((_NWORK, 16), jnp.float32),
        mesh=mesh,
        scratch_types=[
            pltpu.VMEM((_DAYS, _W), jnp.float32),
            pltpu.VMEM((_DAYS, _W), jnp.float32),
            pltpu.VMEM((16,), jnp.float32),
        ],
        compiler_params=pltpu.CompilerParams(use_tc_tiling_on_sc=False),
    )
    def body(o_hbm, t_hbm, out_hbm, pbuf, tbuf, obuf):
        wid = lax.axis_index("s") * 2 + lax.axis_index("c")
        lane = lax.broadcasted_iota(jnp.int32, (16,), 0)
        zero16 = jnp.zeros((16,), jnp.float32)

        def task_body(k, sq):
            task = wid + _NWORK * k
            validf = jnp.where(task < _NTASK, jnp.float32(1.0), jnp.float32(0.0))
            taskc = jnp.minimum(task, _NTASK - 1)
            year = taskc // _NSTRIP
            strip = taskc - year * _NSTRIP
            row0 = year * _DAYS
            col0 = strip * _W
            pltpu.sync_copy(o_hbm.at[pl.ds(row0, _DAYS), pl.ds(col0, _W)], pbuf)
            pltpu.sync_copy(t_hbm.at[pl.ds(row0, _DAYS), pl.ds(col0, _W)], tbuf)
            for g in range(_W // 16):
                def day_body(r, acc):
                    base = r * _RCHUNK
                    for u in range(_RCHUNK):
                        acc = acc + (pbuf[base + u, pl.ds(g * 16, 16)]
                                     - tbuf[base + u, pl.ds(g * 16, 16)])
                    return acc
                s = lax.fori_loop(0, _DAYS // _RCHUNK, day_body, zero16)
                rem = (col0 + g * 16 + lane) % _NCH
                maskf = (1 - jnp.minimum(rem, 1)).astype(jnp.float32)
                sq = sq + s * s * (maskf * validf)
            return sq

        sq = lax.fori_loop(0, _TPW, task_body, zero16)
        obuf[...] = sq
        pltpu.sync_copy(obuf, out_hbm.at[wid])

    return body(o2, t2)


def kernel(output, target):
    nt, ngage, nchan = output.shape
    o2 = output.reshape(nt, ngage * nchan)
    t2 = target.reshape(nt, ngage * nchan)
    partials = _sc_partials(o2, t2)
    scale = 1.0 / (float(_DAYS) * float(_DAYS) * float(_NY) * float(_NG))
    return jnp.sum(partials) * scale


# speedup vs baseline: 1.0437x; 1.0437x over previous
import functools

import jax
import jax.numpy as jnp
from jax import lax
from jax.experimental import pallas as pl
from jax.experimental.pallas import tpu as pltpu
from jax.experimental.pallas import tpu_sc as plsc

_NT = 7300
_NG = 2000
_NCH = 3
_DAYS = 365
_NY = _NT // _DAYS
_COLS = _NG * _NCH
_W = 400
_NGRP = _W // 16
_NSTRIP = _COLS // _W
_NTASK = _NY * _NSTRIP
_NWORK = 32
_TPW = -(-_NTASK // _NWORK)
_DB = 73
_NB = _DAYS // _DB
_RU = 5


def _sc_partials(o2, t2):
    mesh = plsc.VectorSubcoreMesh(core_axis_name="c", subcore_axis_name="s")

    @functools.partial(
        pl.kernel,
        out_type=jax.ShapeDtypeStruct((_NWORK, 16), jnp.float32),
        mesh=mesh,
        scratch_types=[
            pltpu.VMEM((2, _DB, _W), jnp.float32),
            pltpu.VMEM((2, _DB, _W), jnp.float32),
            pltpu.VMEM((_W,), jnp.float32),
            pltpu.VMEM((16,), jnp.float32),
            pltpu.SemaphoreType.DMA,
            pltpu.SemaphoreType.DMA,
            pltpu.SemaphoreType.DMA,
            pltpu.SemaphoreType.DMA,
        ],
        compiler_params=pltpu.CompilerParams(use_tc_tiling_on_sc=False),
    )
    def body(o_hbm, t_hbm, out_hbm, pbuf, tbuf, acc, obuf, s0, s1, s2, s3):
        wid = lax.axis_index("s") * 2 + lax.axis_index("c")
        lane = lax.broadcasted_iota(jnp.int32, (16,), 0)
        zero16 = jnp.zeros((16,), jnp.float32)
        sems = ((s0, s1), (s2, s3))

        def start_block(row0, col0, b, slot):
            po = pltpu.make_async_copy(
                o_hbm.at[pl.ds(row0 + b * _DB, _DB), pl.ds(col0, _W)],
                pbuf.at[slot], sems[slot][0])
            to = pltpu.make_async_copy(
                t_hbm.at[pl.ds(row0 + b * _DB, _DB), pl.ds(col0, _W)],
                tbuf.at[slot], sems[slot][1])
            po.start()
            to.start()
            return po, to

        def task_body(k, sq):
            task = wid + _NWORK * k
            validf = jnp.where(task < _NTASK, jnp.float32(1.0), jnp.float32(0.0))
            taskc = jnp.minimum(task, _NTASK - 1)
            year = taskc // _NSTRIP
            strip = taskc - year * _NSTRIP
            row0 = year * _DAYS
            col0 = strip * _W

            def zero_g(g, c):
                acc[pl.ds(g * 16, 16)] = zero16
                return c
            lax.fori_loop(0, _NGRP, zero_g, 0)

            pending = start_block(row0, col0, 0, 0)
            for b in range(_NB):
                slot = b % 2
                pending[0].wait()
                pending[1].wait()
                if b + 1 < _NB:
                    pending = start_block(row0, col0, b + 1, 1 - slot)
                pcur = pbuf.at[slot]
                tcur = tbuf.at[slot]

                def group_body(g, c):
                    g16 = g * 16

                    def day5(r, reg):
                        base = r * _RU
                        for u in range(_RU):
                            reg = reg + (pcur[base + u, pl.ds(g16, 16)]
                                         - tcur[base + u, pl.ds(g16, 16)])
                        return reg
                    reg = lax.fori_loop(0, _DB // _RU, day5, zero16)
                    for rr in range(_DB - (_DB // _RU) * _RU):
                        row = (_DB // _RU) * _RU + rr
                        reg = reg + (pcur[row, pl.ds(g16, 16)]
                                     - tcur[row, pl.ds(g16, 16)])
                    acc[pl.ds(g16, 16)] = acc[pl.ds(g16, 16)] + reg
                    return c
                lax.fori_loop(0, _NGRP, group_body, 0)

            def square_g(g, s_in):
                s = acc[pl.ds(g * 16, 16)]
                rem = (col0 + g * 16 + lane) % _NCH
                maskf = (1 - jnp.minimum(rem, 1)).astype(jnp.float32)
                return s_in + s * s * (maskf * validf)
            return lax.fori_loop(0, _NGRP, square_g, sq)

        sq = lax.fori_loop(0, _TPW, task_body, zero16)
        obuf[...] = sq
        pltpu.sync_copy(obuf, out_hbm.at[wid])

    return body(o2, t2)


def kernel(output, target):
    nt, ngage, nchan = output.shape
    o2 = output.reshape(nt, ngage * nchan)
    t2 = target.reshape(nt, ngage * nchan)
    partials = _sc_partials(o2, t2)
    scale = 1.0 / (float(_DAYS) * float(_DAYS) * float(_NY) * float(_NG))
    return jnp.sum(partials) * scale


# speedup vs baseline: 1.6350x; 1.5665x over previous
import functools

import jax
import jax.numpy as jnp
from jax import lax
from jax.experimental import pallas as pl
from jax.experimental.pallas import tpu as pltpu
from jax.experimental.pallas import tpu_sc as plsc

_NT = 7300
_NG = 2000
_NCH = 3
_DAYS = 365
_NY = _NT // _DAYS
_COLS = _NG * _NCH
_WW = 256
_NW_CH = 23
_WT = _COLS - _NW_CH * _WW
_TCOL = _NW_CH * _WW
_NWIDE = _NY * _NW_CH
_NWORK = 32
_WPW = -(-_NWIDE // _NWORK)
_DB = 73
_NB = _DAYS // _DB
_RMAIN = 80
_RB = 84
_AFLOOR = _NT - _RB
_RTAIL = _NT - (_NT // 8) * 8
_TAIL0 = _NT - _RTAIL
_RU = 5


def _sc_partials(o2, t2):
    mesh = plsc.VectorSubcoreMesh(core_axis_name="c", subcore_axis_name="s")

    @functools.partial(
        pl.kernel,
        out_type=jax.ShapeDtypeStruct((_NWORK, 8, 16), jnp.float32),
        mesh=mesh,
        scratch_types=[
            pltpu.VMEM((2, _RB, _WW), jnp.float32),
            pltpu.VMEM((2, _RB, _WW), jnp.float32),
            pltpu.VMEM((1, _RB, _WT), jnp.float32),
            pltpu.VMEM((1, _RB, _WT), jnp.float32),
            pltpu.VMEM((_WW,), jnp.float32),
            pltpu.VMEM((8, 16), jnp.float32),
            pltpu.SemaphoreType.DMA,
            pltpu.SemaphoreType.DMA,
            pltpu.SemaphoreType.DMA,
            pltpu.SemaphoreType.DMA,
        ],
    )
    def body(o_hbm, t_hbm, out_hbm, pw, tw, pt, tt, acc, obuf,
             s0, s1, s2, s3):
        wid = lax.axis_index("s") * 2 + lax.axis_index("c")
        lane = lax.broadcasted_iota(jnp.int32, (16,), 0)
        zero16 = jnp.zeros((16,), jnp.float32)
        sems = ((s0, s1), (s2, s3))

        def start_block(pbuf, tbuf, width, year, col0, b, slot):
            start = year * _DAYS + b * _DB
            a0 = pl.multiple_of(
                jnp.minimum((start // 8) * 8, _AFLOOR), 8)
            handles = []
            for src, dst, sem in ((o_hbm, pbuf, sems[slot][0]),
                                  (t_hbm, tbuf, sems[slot][1])):
                h = pltpu.make_async_copy(
                    src.at[pl.ds(a0, _RMAIN), pl.ds(col0, width)],
                    dst.at[slot, pl.ds(0, _RMAIN)], sem)
                h.start()
                handles.append(h)
                if b == _NB - 1:
                    ht = pltpu.make_async_copy(
                        src.at[pl.ds(_TAIL0, _RTAIL), pl.ds(col0, width)],
                        dst.at[slot, pl.ds(_RMAIN, _RTAIL)], sem)
                    ht.start()
                    handles.append(ht)
            return handles, start - a0

        def do_task(pbuf, tbuf, width, ngrp, year, col0, validf, sq,
                    nslots=2):
            def zero_g(g, c):
                acc[pl.ds(g * 16, 16)] = zero16
                return c
            lax.fori_loop(0, ngrp, zero_g, 0)

            pending = start_block(pbuf, tbuf, width, year, col0, 0, 0)
            for b in range(_NB):
                slot = b % nslots
                handles, lead = pending
                for h in handles:
                    h.wait()
                if nslots > 1 and b + 1 < _NB:
                    pending = start_block(pbuf, tbuf, width, year, col0,
                                          b + 1, (b + 1) % nslots)
                pcur = pbuf.at[slot]
                tcur = tbuf.at[slot]

                def group_body(g, c):
                    g16 = g * 16

                    def day5(r, reg):
                        base = lead + r * _RU
                        for u in range(_RU):
                            reg = reg + (pcur[base + u, pl.ds(g16, 16)]
                                         - tcur[base + u, pl.ds(g16, 16)])
                        return reg
                    reg = lax.fori_loop(0, _DB // _RU, day5, zero16)
                    for rr in range(_DB - (_DB // _RU) * _RU):
                        row = lead + (_DB // _RU) * _RU + rr
                        reg = reg + (pcur[row, pl.ds(g16, 16)]
                                     - tcur[row, pl.ds(g16, 16)])
                    acc[pl.ds(g16, 16)] = acc[pl.ds(g16, 16)] + reg
                    return c
                lax.fori_loop(0, ngrp, group_body, 0)
                if nslots == 1 and b + 1 < _NB:
                    pending = start_block(pbuf, tbuf, width, year, col0,
                                          b + 1, 0)

            def square_g(g, s_in):
                s = acc[pl.ds(g * 16, 16)]
                rem = (col0 + g * 16 + lane) % _NCH
                maskf = (1 - jnp.minimum(rem, 1)).astype(jnp.float32)
                return s_in + s * s * (maskf * validf)
            return lax.fori_loop(0, ngrp, square_g, sq)

        def wide_body(k, sq):
            task = wid + _NWORK * k
            validf = jnp.where(task < _NWIDE, jnp.float32(1.0),
                               jnp.float32(0.0))
            taskc = jnp.minimum(task, _NWIDE - 1)
            year = taskc // _NW_CH
            chunk = taskc - year * _NW_CH
            return do_task(pw, tw, _WW, _WW // 16, year, chunk * _WW,
                           validf, sq)

        sq = lax.fori_loop(0, _WPW, wide_body, zero16)

        validf = jnp.where(wid >= _NWORK - _NY, jnp.float32(1.0),
                           jnp.float32(0.0))
        year = jnp.clip(wid - (_NWORK - _NY), 0, _NY - 1)
        sq = do_task(pt, tt, _WT, _WT // 16, year, _TCOL, validf, sq,
                     nslots=1)

        def zero_o(r, c):
            obuf[r] = zero16
            return c
        lax.fori_loop(0, 8, zero_o, 0)
        obuf[0] = sq
        pltpu.sync_copy(obuf, out_hbm.at[wid])

    return body(o2, t2)


def kernel(output, target):
    nt, ngage, nchan = output.shape
    o2 = output.reshape(nt, ngage * nchan)
    t2 = target.reshape(nt, ngage * nchan)
    partials = _sc_partials(o2, t2)
    scale = 1.0 / (float(_DAYS) * float(_DAYS) * float(_NY) * float(_NG))
    return jnp.sum(partials) * scale
